# baseline (speedup 1.0000x reference)
import functools

import jax
import jax.numpy as jnp
import numpy as np
from jax import lax
from jax.experimental import pallas as pl
from jax.experimental.pallas import tpu as pltpu
from jax.experimental.pallas import tpu_sc as plsc


def _half_perms(width):
    lo, hi = [], []
    for g in range(width // 32):
        lo.append(g * 32 + np.arange(16))
        hi.append(g * 32 + 16 + np.arange(16))
    return np.concatenate(lo + hi)


def _pack_bf16_pairs(v):
    w = v.shape[1] // 2
    blo = jax.lax.bitcast_convert_type(v[:, :w], jnp.int32)
    bhi = jax.lax.bitcast_convert_type(v[:, w:], jnp.int32)
    rlo = (blo + 0x7FFF + ((blo >> 16) & 1)) >> 16
    rhi = (bhi + 0x7FFF + ((bhi >> 16) & 1)) >> 16
    return (rhi << 16) | (rlo & 0xFFFF)


def _enc_tables_body(x_ref, h_ref, wex_ref, weh_ref, be_ref, ws_ref, wd_ref,
                     z_ref, a_ref, b_ref):
    z = x_ref[...] @ wex_ref[...] + h_ref[...] @ weh_ref[...] + be_ref[...]
    z = jnp.maximum(z, 0.0)
    z_ref[...] = z
    a_ref[...] = z @ ws_ref[...]
    b_ref[...] = z @ wd_ref[...]


def _edge_proj_body(ea_ref, we_ref, bm_ref, c_ref):
    c_ref[...] = _pack_bf16_pairs(ea_ref[...] @ we_ref[...] + bm_ref[...])


def _decode_body(z_ref, p0_ref, p1_ref, wuz_ref, wua_ref, bu_ref,
                 wdz_ref, wdh_ref, bd_ref, wt_ref, bt_ref,
                 y_ref, tau_ref, hn_ref):
    z = z_ref[...]
    agg = p0_ref[0] + p1_ref[0]
    hn = jnp.maximum(z @ wuz_ref[...] + agg @ wua_ref[...] + bu_ref[...], 0.0)
    hn_ref[...] = hn
    y_ref[...] = z @ wdz_ref[...] + hn @ wdh_ref[...] + bd_ref[...]
    t = hn @ wt_ref[...] + bt_ref[...]
    tau_ref[...] = 1.0 / (1.0 + jnp.exp(-t))


def _make_sc_edge(n_nodes, n_edges, dh):
    NC, NS = 2, 16
    NW = NC * NS
    dw = dh // 2
    epw = n_edges // NW
    K = 40
    CPB = 50
    EPB = K * CPB
    nblk = epw // EPB
    rpt = -(-n_nodes // (32 * NS)) * 32
    n_pad = rpt * NS
    nz = rpt // K

    mesh = plsc.VectorSubcoreMesh(core_axis_name="c", subcore_axis_name="s",
                                  num_cores=NC, num_subcores=NS)

    @functools.partial(
        pl.kernel,
        out_type=jax.ShapeDtypeStruct((NC, n_pad, dh), jnp.float32),
        mesh=mesh,
        scratch_types=[
            pltpu.VMEM((EPB,), jnp.int32),
            pltpu.VMEM((EPB,), jnp.int32),
            [pltpu.VMEM((K,), jnp.int32) for _ in range(2)],
            [pltpu.VMEM((K, dh), jnp.float32) for _ in range(2)],
            [pltpu.VMEM((K, dh), jnp.float32) for _ in range(2)],
            [pltpu.VMEM((K, dw), jnp.int32) for _ in range(2)],
            [pltpu.VMEM((K, dh), jnp.float32) for _ in range(2)],
            pltpu.VMEM_SHARED((n_pad, dh), jnp.float32),
            [pltpu.SemaphoreType.DMA for _ in range(8)],
        ],
    )
    def sc_edge(a_hbm, b_hbm, c_hbm, src_hbm, dst_hbm, out_hbm,
                sidx, didx, dc, ar, br, cr, mr, agg_sh, sems):
        c = lax.axis_index("c")
        s = lax.axis_index("s")
        wid = s * NC + c
        sem_a = sems[0:2]
        sem_b = sems[2:4]
        sem_c = sems[4:6]
        sem_s = sems[6:8]

        def zrow(i, carry):
            for j in range(dh // 16):
                mr[0][i, pl.ds(j * 16, 16)] = jnp.zeros((16,), jnp.float32)
            return carry
        lax.fori_loop(0, K, zrow, 0)
        zcps = [pltpu.make_async_copy(
            mr[0], agg_sh.at[pl.ds(s * rpt + t * K, K)], sems[6])
            for t in range(nz)]
        for cp in zcps:
            cp.start()
        for cp in zcps:
            cp.wait()
        plsc.subcore_barrier()

        def gather_cps(cc, p, base_e):
            isl = sidx.at[pl.ds(cc * K, K)]
            dsl = didx.at[pl.ds(cc * K, K)]
            return (
                pltpu.make_async_copy(a_hbm.at[isl], ar[p], sem_a[p]),
                pltpu.make_async_copy(b_hbm.at[dsl], br[p], sem_b[p]),
                pltpu.make_async_copy(
                    c_hbm.at[pl.ds(base_e + cc * K, K)], cr[p], sem_c[p]),
            )

        def g_issue(cc, p, base_e):
            for cp in gather_cps(cc, p, base_e):
                cp.start()

        def g_wait(cc, p, base_e):
            for cp in gather_cps(cc, p, base_e):
                cp.wait()

        def prep_didx(cc, p):
            off = cc * K
            dc[p][pl.ds(0, 16)] = didx[pl.ds(off, 16)]
            dc[p][pl.ds(16, 16)] = didx[pl.ds(off + 16, 16)]
            dc[p][pl.ds(24, 16)] = didx[pl.ds(off + 24, 16)]

        def compute(p):
            zero16 = jnp.zeros((16,), jnp.float32)
            himask = jnp.full((16,), -65536, jnp.int32)

            def as_f32(w):
                lo = lax.bitcast_convert_type(w << 16, jnp.float32)
                hi = lax.bitcast_convert_type(w & himask, jnp.float32)
                return lo, hi

            @plsc.parallel_loop(0, K, unroll=4)
            def _(e):
                for g in range(dw // 16):
                    lo_sl = pl.ds(g * 32, 16)
                    hi_sl = pl.ds(g * 32 + 16, 16)
                    clo, chi = as_f32(cr[p][e, pl.ds(g * 16, 16)])
                    mr[p][e, lo_sl] = jnp.maximum(
                        ar[p][e, lo_sl] + br[p][e, lo_sl] + clo, zero16)
                    mr[p][e, hi_sl] = jnp.maximum(
                        ar[p][e, hi_sl] + br[p][e, hi_sl] + chi, zero16)

        def s_issue(p):
            pltpu.async_copy(mr[p], agg_sh.at[dc[p]], sem_s[p], add=True)

        def s_wait(p):
            pltpu.make_async_copy(mr[p], agg_sh.at[dc[p]], sem_s[p]).wait()

        def block(blk, carry):
            base_e = wid * epw + blk * EPB
            pltpu.sync_copy(src_hbm.at[pl.ds(base_e, EPB)], sidx)
            pltpu.sync_copy(dst_hbm.at[pl.ds(base_e, EPB)], didx)
            g_issue(0, 0, base_e)
            g_issue(1, 1, base_e)
            g_wait(0, 0, base_e)
            prep_didx(0, 0)
            compute(0)
            g_issue(2, 0, base_e)
            s_issue(0)
            g_wait(1, 1, base_e)
            prep_didx(1, 1)
            compute(1)
            g_issue(3, 1, base_e)
            s_issue(1)

            def pair(j, pcarry):
                c0 = 2 * j
                g_wait(c0, 0, base_e)
                s_wait(0)
                prep_didx(c0, 0)
                compute(0)
                g_issue(c0 + 2, 0, base_e)
                s_issue(0)
                g_wait(c0 + 1, 1, base_e)
                s_wait(1)
                prep_didx(c0 + 1, 1)
                compute(1)
                g_issue(c0 + 3, 1, base_e)
                s_issue(1)
                return pcarry
            lax.fori_loop(1, CPB // 2 - 1, pair, 0)

            c0 = CPB - 2
            g_wait(c0, 0, base_e)
            s_wait(0)
            prep_didx(c0, 0)
            compute(0)
            s_issue(0)
            g_wait(c0 + 1, 1, base_e)
            s_wait(1)
            prep_didx(c0 + 1, 1)
            compute(1)
            s_issue(1)
            s_wait(0)
            s_wait(1)
            return carry

        lax.fori_loop(0, nblk, block, 0)
        plsc.subcore_barrier()

        rcps = [pltpu.make_async_copy(
            agg_sh.at[pl.ds(s * rpt + t * K, K)],
            out_hbm.at[c, pl.ds(s * rpt + t * K, K)], sems[6])
            for t in range(nz)]
        for cp in rcps:
            cp.start()
        for cp in rcps:
            cp.wait()

    return sc_edge


@jax.jit
def kernel(x, h, edge_attr, W_enc, b_enc, W_msg, b_msg, W_upd, b_upd,
           W_dec, b_dec, W_term, b_term, edge_index):
    n, d_in = x.shape
    dh = h.shape[1]
    e, de = edge_attr.shape
    d_out = W_dec.shape[1]

    src = edge_index[0]
    dst = edge_index[1]
    perm = jnp.asarray(_half_perms(dh))
    wex = W_enc[:d_in]
    weh = W_enc[d_in:]
    ws = W_msg[:dh]
    wd = W_msg[dh:2 * dh]
    we = W_msg[2 * dh:][:, perm]
    bm = b_msg[perm]
    wuz = W_upd[:dh]
    wua = W_upd[dh:]
    wdz = W_dec[:dh]
    wdh = W_dec[dh:]

    bn = 1000
    gn = n // bn
    be = 2000
    ge = e // be
    dw = dh // 2

    full = lambda shp: pl.BlockSpec(shp, lambda i: (0,) * len(shp))
    rows = lambda w: pl.BlockSpec((bn, w), lambda i: (i, 0))

    z, a_tab, b_tab = pl.pallas_call(
        _enc_tables_body,
        grid=(gn,),
        in_specs=[
            rows(d_in), rows(dh),
            full((d_in, dh)), full((dh, dh)), full((1, dh)),
            full((dh, dh)), full((dh, dh)),
        ],
        out_specs=[rows(dh), rows(dh), rows(dh)],
        out_shape=[
            jax.ShapeDtypeStruct((n, dh), jnp.float32),
            jax.ShapeDtypeStruct((n, dh), jnp.float32),
            jax.ShapeDtypeStruct((n, dh), jnp.float32),
        ],
    )(x, h, wex, weh, b_enc.reshape(1, dh), ws, wd)

    c_tab = pl.pallas_call(
        _edge_proj_body,
        grid=(ge,),
        in_specs=[
            pl.BlockSpec((be, de), lambda i: (i, 0)),
            full((de, dh)), full((1, dh)),
        ],
        out_specs=pl.BlockSpec((be, dw), lambda i: (i, 0)),
        out_shape=jax.ShapeDtypeStruct((e, dw), jnp.int32),
    )(edge_attr, we, bm.reshape(1, dh))

    sc_edge = _make_sc_edge(n, e, dh)
    parts = sc_edge(a_tab, b_tab, c_tab, src, dst)

    y, tau, h_new = pl.pallas_call(
        _decode_body,
        grid=(gn,),
        in_specs=[
            rows(dh),
            pl.BlockSpec((1, bn, dh), lambda i: (0, i, 0)),
            pl.BlockSpec((1, bn, dh), lambda i: (1, i, 0)),
            full((dh, dh)), full((dh, dh)), full((1, dh)),
            full((dh, d_out)), full((dh, d_out)), full((1, d_out)),
            full((dh, 1)), full((1, 1)),
        ],
        out_specs=[rows(d_out), pl.BlockSpec((bn, 1), lambda i: (i, 0)),
                   rows(dh)],
        out_shape=[
            jax.ShapeDtypeStruct((n, d_out), jnp.float32),
            jax.ShapeDtypeStruct((n, 1), jnp.float32),
            jax.ShapeDtypeStruct((n, dh), jnp.float32),
        ],
    )(z, parts, parts, wuz, wua, b_upd.reshape(1, dh),
      wdz, wdh, b_dec.reshape(1, d_out), W_term, b_term.reshape(1, 1))

    return (y, tau, h_new)


# speedup vs baseline: 1.1751x; 1.1751x over previous
import functools

import jax
import jax.numpy as jnp
from jax import lax
from jax.experimental import pallas as pl
from jax.experimental.pallas import tpu as pltpu
from jax.experimental.pallas import tpu_sc as plsc


def _enc_tables_body(x_ref, h_ref, wex_ref, weh_ref, be_ref, ws_ref, wd_ref,
                     z_ref, a_ref, b_ref):
    z = x_ref[...] @ wex_ref[...] + h_ref[...] @ weh_ref[...] + be_ref[...]
    z = jnp.maximum(z, 0.0)
    z_ref[...] = z
    a_ref[...] = z @ ws_ref[...]
    b_ref[...] = z @ wd_ref[...]


def _edge_proj_body(ea_ref, we_ref, bm_ref, c_ref):
    c_ref[...] = ea_ref[...] @ we_ref[...] + bm_ref[...]


def _decode_body(z_ref, p0_ref, p1_ref, wuz_ref, wua_ref, bu_ref,
                 wdz_ref, wdh_ref, bd_ref, wt_ref, bt_ref,
                 y_ref, tau_ref, hn_ref):
    z = z_ref[...]
    agg = p0_ref[0] + p1_ref[0]
    hn = jnp.maximum(z @ wuz_ref[...] + agg @ wua_ref[...] + bu_ref[...], 0.0)
    hn_ref[...] = hn
    y_ref[...] = z @ wdz_ref[...] + hn @ wdh_ref[...] + bd_ref[...]
    t = hn @ wt_ref[...] + bt_ref[...]
    tau_ref[...] = 1.0 / (1.0 + jnp.exp(-t))


def _make_sc_edge(n_nodes, n_edges, dh):
    NC, NS = 2, 16
    NW = NC * NS
    epw = n_edges // NW
    K = 40
    CPB = 50
    EPB = K * CPB
    nblk = epw // EPB
    rpt = -(-n_nodes // (32 * NS)) * 32
    n_pad = rpt * NS
    nz = rpt // K

    mesh = plsc.VectorSubcoreMesh(core_axis_name="c", subcore_axis_name="s",
                                  num_cores=NC, num_subcores=NS)

    @functools.partial(
        pl.kernel,
        out_type=jax.ShapeDtypeStruct((NC, n_pad, dh), jnp.float32),
        mesh=mesh,
        scratch_types=[
            pltpu.VMEM((EPB,), jnp.int32),
            pltpu.VMEM((EPB,), jnp.int32),
            [pltpu.VMEM((K,), jnp.int32) for _ in range(2)],
            [pltpu.VMEM((K, dh), jnp.float32) for _ in range(2)],
            [pltpu.VMEM((K, dh), jnp.float32) for _ in range(2)],
            [pltpu.VMEM((K, dh), jnp.float32) for _ in range(2)],
            [pltpu.VMEM((K, dh), jnp.float32) for _ in range(2)],
            pltpu.VMEM_SHARED((n_pad, dh), jnp.float32),
            [pltpu.SemaphoreType.DMA for _ in range(8)],
        ],
    )
    def sc_edge(a_hbm, b_hbm, c_hbm, src_hbm, dst_hbm, out_hbm,
                sidx, didx, dc, ar, br, cr, mr, agg_sh, sems):
        c = lax.axis_index("c")
        s = lax.axis_index("s")
        wid = s * NC + c
        sem_a = sems[0:2]
        sem_b = sems[2:4]
        sem_c = sems[4:6]
        sem_s = sems[6:8]

        def zrow(i, carry):
            for j in range(dh // 16):
                ar[0][i, pl.ds(j * 16, 16)] = jnp.zeros((16,), jnp.float32)
            return carry
        lax.fori_loop(0, K, zrow, 0)
        zcps = [pltpu.make_async_copy(
            ar[0], agg_sh.at[pl.ds(s * rpt + t * K, K)], sems[6])
            for t in range(nz)]
        for cp in zcps:
            cp.start()
        for cp in zcps:
            cp.wait()
        plsc.subcore_barrier()

        def gather_cps(cc, p, base_e):
            isl = sidx.at[pl.ds(cc * K, K)]
            dsl = didx.at[pl.ds(cc * K, K)]
            return (
                pltpu.make_async_copy(a_hbm.at[isl], ar[p], sem_a[p]),
                pltpu.make_async_copy(b_hbm.at[dsl], br[p], sem_b[p]),
                pltpu.make_async_copy(
                    c_hbm.at[pl.ds(base_e + cc * K, K)], cr[p], sem_c[p]),
            )

        def g_issue(cc, p, base_e):
            for cp in gather_cps(cc, p, base_e):
                cp.start()

        def g_wait(cc, p, base_e):
            for cp in gather_cps(cc, p, base_e):
                cp.wait()

        def prep_didx(cc, p):
            off = cc * K
            dc[p][pl.ds(0, 16)] = didx[pl.ds(off, 16)]
            dc[p][pl.ds(16, 16)] = didx[pl.ds(off + 16, 16)]
            dc[p][pl.ds(24, 16)] = didx[pl.ds(off + 24, 16)]

        def compute(p):
            @plsc.parallel_loop(0, K, unroll=4)
            def _(e):
                for j in range(dh // 16):
                    sl = pl.ds(j * 16, 16)
                    v = ar[p][e, sl] + br[p][e, sl] + cr[p][e, sl]
                    mr[p][e, sl] = jnp.maximum(v, 0.0)

        def s_issue(p):
            pltpu.async_copy(mr[p], agg_sh.at[dc[p]], sem_s[p], add=True)

        def s_wait(p):
            pltpu.make_async_copy(mr[p], agg_sh.at[dc[p]], sem_s[p]).wait()

        def block(blk, carry):
            base_e = wid * epw + blk * EPB
            pltpu.sync_copy(src_hbm.at[pl.ds(base_e, EPB)], sidx)
            pltpu.sync_copy(dst_hbm.at[pl.ds(base_e, EPB)], didx)
            g_issue(0, 0, base_e)
            g_issue(1, 1, base_e)
            g_wait(0, 0, base_e)
            prep_didx(0, 0)
            compute(0)
            g_issue(2, 0, base_e)
            s_issue(0)
            g_wait(1, 1, base_e)
            prep_didx(1, 1)
            compute(1)
            g_issue(3, 1, base_e)
            s_issue(1)

            def pair(j, pcarry):
                c0 = 2 * j
                g_wait(c0, 0, base_e)
                s_wait(0)
                prep_didx(c0, 0)
                compute(0)
                g_issue(c0 + 2, 0, base_e)
                s_issue(0)
                g_wait(c0 + 1, 1, base_e)
                s_wait(1)
                prep_didx(c0 + 1, 1)
                compute(1)
                g_issue(c0 + 3, 1, base_e)
                s_issue(1)
                return pcarry
            lax.fori_loop(1, CPB // 2 - 1, pair, 0)

            c0 = CPB - 2
            g_wait(c0, 0, base_e)
            s_wait(0)
            prep_didx(c0, 0)
            compute(0)
            s_issue(0)
            g_wait(c0 + 1, 1, base_e)
            s_wait(1)
            prep_didx(c0 + 1, 1)
            compute(1)
            s_issue(1)
            s_wait(0)
            s_wait(1)
            return carry

        lax.fori_loop(0, nblk, block, 0)
        plsc.subcore_barrier()

        rcps = [pltpu.make_async_copy(
            agg_sh.at[pl.ds(s * rpt + t * K, K)],
            out_hbm.at[c, pl.ds(s * rpt + t * K, K)], sems[6])
            for t in range(nz)]
        for cp in rcps:
            cp.start()
        for cp in rcps:
            cp.wait()

    return sc_edge


@jax.jit
def kernel(x, h, edge_attr, W_enc, b_enc, W_msg, b_msg, W_upd, b_upd,
           W_dec, b_dec, W_term, b_term, edge_index):
    n, d_in = x.shape
    dh = h.shape[1]
    e, de = edge_attr.shape
    d_out = W_dec.shape[1]

    src = edge_index[0]
    dst = edge_index[1]
    wex = W_enc[:d_in]
    weh = W_enc[d_in:]
    ws = W_msg[:dh]
    wd = W_msg[dh:2 * dh]
    we = W_msg[2 * dh:]
    wuz = W_upd[:dh]
    wua = W_upd[dh:]
    wdz = W_dec[:dh]
    wdh = W_dec[dh:]

    bn = 2000
    gn = n // bn
    be = 16000
    ge = e // be

    full = lambda shp: pl.BlockSpec(shp, lambda i: (0,) * len(shp))
    rows = lambda w: pl.BlockSpec((bn, w), lambda i: (i, 0))

    z, a_tab, b_tab = pl.pallas_call(
        _enc_tables_body,
        grid=(gn,),
        in_specs=[
            rows(d_in), rows(dh),
            full((d_in, dh)), full((dh, dh)), full((1, dh)),
            full((dh, dh)), full((dh, dh)),
        ],
        out_specs=[rows(dh), rows(dh), rows(dh)],
        out_shape=[
            jax.ShapeDtypeStruct((n, dh), jnp.float32),
            jax.ShapeDtypeStruct((n, dh), jnp.float32),
            jax.ShapeDtypeStruct((n, dh), jnp.float32),
        ],
    )(x, h, wex, weh, b_enc.reshape(1, dh), ws, wd)

    c_tab = pl.pallas_call(
        _edge_proj_body,
        grid=(ge,),
        in_specs=[
            pl.BlockSpec((be, de), lambda i: (i, 0)),
            full((de, dh)), full((1, dh)),
        ],
        out_specs=pl.BlockSpec((be, dh), lambda i: (i, 0)),
        out_shape=jax.ShapeDtypeStruct((e, dh), jnp.float32),
    )(edge_attr, we, b_msg.reshape(1, dh))

    sc_edge = _make_sc_edge(n, e, dh)
    parts = sc_edge(a_tab, b_tab, c_tab, src, dst)

    y, tau, h_new = pl.pallas_call(
        _decode_body,
        grid=(gn,),
        in_specs=[
            rows(dh),
            pl.BlockSpec((1, bn, dh), lambda i: (0, i, 0)),
            pl.BlockSpec((1, bn, dh), lambda i: (1, i, 0)),
            full((dh, dh)), full((dh, dh)), full((1, dh)),
            full((dh, d_out)), full((dh, d_out)), full((1, d_out)),
            full((dh, 1)), full((1, 1)),
        ],
        out_specs=[rows(d_out), pl.BlockSpec((bn, 1), lambda i: (i, 0)),
                   rows(dh)],
        out_shape=[
            jax.ShapeDtypeStruct((n, d_out), jnp.float32),
            jax.ShapeDtypeStruct((n, 1), jnp.float32),
            jax.ShapeDtypeStruct((n, dh), jnp.float32),
        ],
    )(z, parts, parts, wuz, wua, b_upd.reshape(1, dh),
      wdz, wdh, b_dec.reshape(1, d_out), W_term, b_term.reshape(1, 1))

    return (y, tau, h_new)
